# baseline (speedup 1.0000x reference)
import functools

import jax
import jax.numpy as jnp
from jax import lax
from jax.experimental import pallas as pl
from jax.experimental.pallas import tpu as pltpu
from jax.experimental.pallas import tpu_sc as plsc

_N_GRAPH, _N_NODE, _D = 256, 128, 512
_NC, _NS = 2, 16
_NW = _NC * _NS
_ROWS = _N_GRAPH // _NW
_L = 16


def _gather_body(xf, root, out, root_v, idx_v, rows_v, sem):
    s = lax.axis_index("s")
    c = lax.axis_index("c")
    wid = s * _NC + c
    pltpu.sync_copy(root.at[pl.ds(s * _L, _L)], root_v)
    idx_v[...] = root_v[...] + (lax.iota(jnp.int32, _L) + s * _L) * _N_NODE
    cp = pltpu.async_copy(xf.at[idx_v.at[pl.ds(c * _ROWS, _ROWS)]], rows_v, sem)
    cp.wait()
    pltpu.sync_copy(rows_v, out.at[pl.ds(wid * _ROWS, _ROWS)])


def kernel(x, x_mask, root_n_id, attn):
    del x_mask, attn
    xf = x.reshape(-1, _D)
    root = root_n_id.astype(jnp.int32)
    mesh = plsc.VectorSubcoreMesh(core_axis_name="c", subcore_axis_name="s")
    f = functools.partial(
        pl.kernel,
        mesh=mesh,
        out_type=jax.ShapeDtypeStruct((_N_GRAPH, _D), jnp.float32),
        scratch_types=[
            pltpu.VMEM((_L,), jnp.int32),
            pltpu.VMEM((_L,), jnp.int32),
            pltpu.VMEM((_ROWS, _D), jnp.float32),
            pltpu.SemaphoreType.DMA,
        ],
    )(_gather_body)
    return f(xf, root)


# speedup vs baseline: 1.0676x; 1.0676x over previous
import functools

import jax
import jax.numpy as jnp
from jax import lax
from jax.experimental import pallas as pl
from jax.experimental.pallas import tpu as pltpu
from jax.experimental.pallas import tpu_sc as plsc

_N_GRAPH, _N_NODE, _D = 256, 128, 512
_NS = 16
_ROWS = _N_GRAPH // _NS
_L = 16


def _gather_body(xf, root, out, root_v, idx_v, rows_v, sem):
    s = lax.axis_index("s")
    base = s * _ROWS
    pltpu.sync_copy(root.at[pl.ds(base, _ROWS)], root_v)
    idx_v[...] = root_v[...] + (lax.iota(jnp.int32, _L) + base) * _N_NODE
    cp = pltpu.async_copy(xf.at[idx_v], rows_v, sem)
    cp.wait()
    pltpu.sync_copy(rows_v, out.at[pl.ds(base, _ROWS)])


def kernel(x, x_mask, root_n_id, attn):
    del x_mask, attn
    xf = x.reshape(-1, _D)
    root = root_n_id.astype(jnp.int32)
    mesh = plsc.VectorSubcoreMesh(
        core_axis_name="c", subcore_axis_name="s", num_cores=1
    )
    f = functools.partial(
        pl.kernel,
        mesh=mesh,
        out_type=jax.ShapeDtypeStruct((_N_GRAPH, _D), jnp.float32),
        scratch_types=[
            pltpu.VMEM((_L,), jnp.int32),
            pltpu.VMEM((_L,), jnp.int32),
            pltpu.VMEM((_ROWS, _D), jnp.float32),
            pltpu.SemaphoreType.DMA,
        ],
    )(_gather_body)
    return f(xf, root)
